# baseline (speedup 1.0000x reference)
import functools

import jax
import jax.numpy as jnp
from jax.experimental import pallas as pl

N = 50000
E = 800000
D = 64
R = 8
BLK = 1000


def _mm_body(h_ref, w_ref, o_ref):
    h = h_ref[...]
    for r in range(R + 1):
        o_ref[r] = jnp.dot(h, w_ref[r], preferred_element_type=jnp.float32)


def _mm(h, wall):
    grid = (N // BLK,)
    return pl.pallas_call(
        _mm_body,
        grid=grid,
        in_specs=[
            pl.BlockSpec((BLK, D), lambda i: (i, 0)),
            pl.BlockSpec((R + 1, D, D), lambda i: (0, 0, 0)),
        ],
        out_specs=pl.BlockSpec((R + 1, BLK, D), lambda i: (0, i, 0)),
        out_shape=jax.ShapeDtypeStruct((R + 1, N, D), jnp.float32),
    )(h, wall)


def kernel(x, edge_index, edge_type, emb, W1, root1, b1, W2, root2, b2):
    src = edge_index[0].astype(jnp.int32)
    dst = edge_index[1].astype(jnp.int32)
    et = edge_type.astype(jnp.int32)
    seg = dst * R + et
    gidx = et * N + src
    h = jnp.take(emb, x, axis=0)
    cnt = jax.ops.segment_sum(jnp.ones((E,), jnp.float32), seg, num_segments=N * R)
    denom = jnp.maximum(cnt, 1.0)[:, None]
    for (W, root, b, relu) in ((W1, root1, b1, True), (W2, root2, b2, False)):
        wall = jnp.concatenate([W, root[None]], axis=0)
        hall = _mm(h, wall)
        msgs = jnp.take(hall[:R].reshape(R * N, D), gidx, axis=0)
        sums = jax.ops.segment_sum(msgs, seg, num_segments=N * R)
        agg = (sums / denom).reshape(N, R, D).sum(axis=1)
        h = agg + hall[R] + b
        if relu:
            h = jax.nn.relu(h)
    return h


# speedup vs baseline: 1.7012x; 1.7012x over previous
import functools

import jax
import jax.numpy as jnp
from jax import lax
from jax.experimental import pallas as pl
from jax.experimental.pallas import tpu as pltpu
from jax.experimental.pallas import tpu_sc as plsc

N = 50000
E = 800000
D = 64
R = 8

NS = 16
NC = 2
E2 = 819200
EP = E2 // NS
MACRO = 1024
NB = EP // MACRO
BATCH = 128
NBB = MACRO // BATCH
NRP = 400384
NHALF = N // 2
ACC = 25600
DUMP = NHALF
BLK = 1000
CBLK = 200

_mesh = plsc.VectorSubcoreMesh(core_axis_name="c", subcore_axis_name="s")


def _fill(ref, n, val):
    v = jnp.full((16,), val, ref.dtype)
    def body(i, carry):
        ref[pl.ds(i * 16, 16)] = v
        return carry
    lax.fori_loop(0, n // 16, body, 0)


def _hist_body(dst_h, et_h, out_h, cnt_sh, dbuf, tbuf, segbuf, ones, zbuf):
    c = lax.axis_index("c")
    s = lax.axis_index("s")
    _fill(zbuf, 3200, 0.0)
    _fill(ones, BATCH, 1.0)
    for j in range(8):
        pltpu.sync_copy(zbuf.at[pl.ds(0, 3128)],
                        cnt_sh.at[pl.ds(s * 25024 + j * 3128, 3128)])
    plsc.subcore_barrier()

    def macro_body(k, carry):
        base = s * EP + k * MACRO
        pltpu.sync_copy(dst_h.at[pl.ds(base, MACRO)], dbuf)
        pltpu.sync_copy(et_h.at[pl.ds(base, MACRO)], tbuf)
        for b in range(NBB):
            for j in range(BATCH // 16):
                o = b * BATCH + j * 16
                dv = dbuf[pl.ds(o, 16)]
                tv = tbuf[pl.ds(o, 16)]
                segbuf[pl.ds(j * 16, 16)] = dv * R + tv
            pltpu.sync_copy(ones, cnt_sh.at[segbuf], add=True)
        return carry

    lax.fori_loop(0, NB, macro_body, 0)
    plsc.subcore_barrier()
    half = NRP // 2
    for j in range(4):
        off = c * half + s * 12512 + j * 3128
        pltpu.sync_copy(cnt_sh.at[pl.ds(off, 3128)], zbuf.at[pl.ds(0, 3128)])
        pltpu.sync_copy(zbuf.at[pl.ds(0, 3128)], out_h.at[pl.ds(off, 3128)])


_hist = functools.partial(
    pl.kernel,
    out_type=jax.ShapeDtypeStruct((NRP,), jnp.float32),
    mesh=_mesh,
    compiler_params=pltpu.CompilerParams(needs_layout_passes=False, use_tc_tiling_on_sc=False),
    scratch_types=[
        pltpu.VMEM_SHARED((NRP,), jnp.float32),
        pltpu.VMEM((MACRO,), jnp.int32),
        pltpu.VMEM((MACRO,), jnp.int32),
        pltpu.VMEM((BATCH,), jnp.int32),
        pltpu.VMEM((BATCH,), jnp.float32),
        pltpu.VMEM((3200,), jnp.float32),
    ],
)(_hist_body)


def _edge_body(src_h, et_h, dst_h, cnt_h, hflat_h, out_h,
               acc_sh, sbuf, tbuf, dbuf, gbuf, segbuf, lbuf, cbuf, wbuf,
               rows, zbuf):
    c = lax.axis_index("c")
    s = lax.axis_index("s")
    lo = c * NHALF

    def zfill(i, carry):
        for kk in range(4):
            zbuf[i, pl.ds(kk * 16, 16)] = jnp.zeros((16,), jnp.float32)
        return carry
    lax.fori_loop(0, 32, zfill, 0)

    def zcopy(j, carry):
        pltpu.sync_copy(zbuf, acc_sh.at[pl.ds(s * 1600 + j * 32, 32)])
        return carry
    lax.fori_loop(0, 50, zcopy, 0)
    plsc.subcore_barrier()

    def macro_body(k, carry):
        base = s * EP + k * MACRO
        pltpu.sync_copy(src_h.at[pl.ds(base, MACRO)], sbuf)
        pltpu.sync_copy(et_h.at[pl.ds(base, MACRO)], tbuf)
        pltpu.sync_copy(dst_h.at[pl.ds(base, MACRO)], dbuf)
        for b in range(NBB):
            for j in range(BATCH // 16):
                o = b * BATCH + j * 16
                sv = sbuf[pl.ds(o, 16)]
                tv = tbuf[pl.ds(o, 16)]
                dv = dbuf[pl.ds(o, 16)]
                gbuf[pl.ds(j * 16, 16)] = tv * N + sv
                segbuf[pl.ds(j * 16, 16)] = dv * R + tv
                m = (dv >= lo) & (dv < lo + NHALF)
                lbuf[pl.ds(j * 16, 16)] = jnp.where(m, dv - lo, DUMP)
            pltpu.sync_copy(cnt_h.at[segbuf], cbuf)
            for j in range(BATCH // 16):
                cv = cbuf[pl.ds(j * 16, 16)]
                lv = lbuf[pl.ds(j * 16, 16)]
                m = lv < DUMP
                w = 1.0 / jnp.maximum(cv, 1.0)
                wbuf[pl.ds(j * 16, 16)] = jnp.where(m, w, 0.0)
            pltpu.sync_copy(hflat_h.at[gbuf], rows)

            def mul_body(i, carry):
                wv = plsc.load_gather(wbuf, [jnp.full((16,), i, jnp.int32)])
                for kk in range(4):
                    rows[i, pl.ds(kk * 16, 16)] = rows[i, pl.ds(kk * 16, 16)] * wv
                return carry
            lax.fori_loop(0, BATCH, mul_body, 0)
            pltpu.sync_copy(rows, acc_sh.at[lbuf], add=True)
        return carry

    lax.fori_loop(0, NB, macro_body, 0)
    plsc.subcore_barrier()

    def wb(j, carry):
        pltpu.sync_copy(acc_sh.at[pl.ds(s * 1600 + j * 32, 32)], zbuf)
        pltpu.sync_copy(zbuf, out_h.at[pl.ds(c * ACC + s * 1600 + j * 32, 32)])
        return carry
    lax.fori_loop(0, 50, wb, 0)


_edge = functools.partial(
    pl.kernel,
    out_type=jax.ShapeDtypeStruct((2 * ACC, D), jnp.float32),
    mesh=_mesh,
    compiler_params=pltpu.CompilerParams(needs_layout_passes=False, use_tc_tiling_on_sc=False),
    scratch_types=[
        pltpu.VMEM_SHARED((ACC, D), jnp.float32),
        pltpu.VMEM((MACRO,), jnp.int32),
        pltpu.VMEM((MACRO,), jnp.int32),
        pltpu.VMEM((MACRO,), jnp.int32),
        pltpu.VMEM((BATCH,), jnp.int32),
        pltpu.VMEM((BATCH,), jnp.int32),
        pltpu.VMEM((BATCH,), jnp.int32),
        pltpu.VMEM((BATCH,), jnp.float32),
        pltpu.VMEM((BATCH,), jnp.float32),
        pltpu.VMEM((BATCH, D), jnp.float32),
        pltpu.VMEM((32, D), jnp.float32),
    ],
)(_edge_body)


def _mm_body(h_ref, w_ref, o_ref):
    h = h_ref[...]
    for r in range(R + 1):
        o_ref[r] = jnp.dot(h, w_ref[r], preferred_element_type=jnp.float32)


def _mm(h, wall):
    return pl.pallas_call(
        _mm_body,
        grid=(N // BLK,),
        in_specs=[
            pl.BlockSpec((BLK, D), lambda i: (i, 0)),
            pl.BlockSpec((R + 1, D, D), lambda i: (0, 0, 0)),
        ],
        out_specs=pl.BlockSpec((R + 1, BLK, D), lambda i: (0, i, 0)),
        out_shape=jax.ShapeDtypeStruct((R + 1, N, D), jnp.float32),
    )(h, wall)


def _combine_body_relu(a_ref, r_ref, b_ref, o_ref):
    o_ref[...] = jnp.maximum(a_ref[...] + r_ref[0] + b_ref[...], 0.0)


def _combine_body(a_ref, r_ref, b_ref, o_ref):
    o_ref[...] = a_ref[...] + r_ref[0] + b_ref[...]


def _combine(aggp, hall, b, relu):
    body = _combine_body_relu if relu else _combine_body
    return pl.pallas_call(
        body,
        grid=(N // CBLK,),
        in_specs=[
            pl.BlockSpec((CBLK, D), lambda i: (jnp.where(i < 125, i, i + 3), 0)),
            pl.BlockSpec((1, CBLK, D), lambda i: (R, i, 0)),
            pl.BlockSpec((1, D), lambda i: (0, 0)),
        ],
        out_specs=pl.BlockSpec((CBLK, D), lambda i: (i, 0)),
        out_shape=jax.ShapeDtypeStruct((N, D), jnp.float32),
    )(aggp, hall, b)


def kernel(x, edge_index, edge_type, emb, W1, root1, b1, W2, root2, b2):
    src = edge_index[0].astype(jnp.int32)
    dst = edge_index[1].astype(jnp.int32)
    et = edge_type.astype(jnp.int32)
    pad = E2 - E
    srcp = jnp.concatenate([src, jnp.zeros((pad,), jnp.int32)])
    etp = jnp.concatenate([et, jnp.zeros((pad,), jnp.int32)])
    dstp = jnp.concatenate([dst, jnp.full((pad,), N, jnp.int32)])
    h = jnp.take(emb, x, axis=0)
    cnt = _hist(dstp, etp)
    for (W, root, b, relu) in ((W1, root1, b1, True), (W2, root2, b2, False)):
        wall = jnp.concatenate([W, root[None]], axis=0)
        hall = _mm(h, wall)
        aggp = _edge(srcp, etp, dstp, cnt, hall.reshape((R + 1) * N, D))
        h = _combine(aggp, hall, b.reshape(1, D), relu)
    return h


# speedup vs baseline: 2.1322x; 1.2534x over previous
import functools

import jax
import jax.numpy as jnp
from jax import lax
from jax.experimental import pallas as pl
from jax.experimental.pallas import tpu as pltpu
from jax.experimental.pallas import tpu_sc as plsc

N = 50000
E = 800000
D = 64
R = 8

NS = 16
NC = 2
E2 = 819200
EP = E2 // NS
MACRO = 256
NB = EP // MACRO
BATCH = 128
NBB = MACRO // BATCH
NRP = 400384
NHALF = N // 2
ACC = 25600
DUMP = NHALF
BLK = 1000
CBLK = 200

_mesh = plsc.VectorSubcoreMesh(core_axis_name="c", subcore_axis_name="s")
_params = pltpu.CompilerParams(needs_layout_passes=False,
                               use_tc_tiling_on_sc=False)


def _fill(ref, n, val):
    v = jnp.full((16,), val, ref.dtype)
    def body(i, carry):
        ref[pl.ds(i * 16, 16)] = v
        return carry
    lax.fori_loop(0, n // 16, body, 0)


def _hist_body(dst_h, et_h, out_h, cnt_sh, dbuf, tbuf, segbuf, ones, zbuf):
    c = lax.axis_index("c")
    s = lax.axis_index("s")
    _fill(zbuf, 3200, 0.0)
    _fill(ones, BATCH, 1.0)
    for j in range(8):
        pltpu.sync_copy(zbuf.at[pl.ds(0, 3128)],
                        cnt_sh.at[pl.ds(s * 25024 + j * 3128, 3128)])
    plsc.subcore_barrier()

    def macro_body(k, carry):
        base = s * EP + k * MACRO
        pltpu.sync_copy(dst_h.at[pl.ds(base, MACRO)], dbuf)
        pltpu.sync_copy(et_h.at[pl.ds(base, MACRO)], tbuf)
        for b in range(NBB):
            for j in range(BATCH // 16):
                o = b * BATCH + j * 16
                dv = dbuf[pl.ds(o, 16)]
                tv = tbuf[pl.ds(o, 16)]
                segbuf[pl.ds(j * 16, 16)] = dv * R + tv
            pltpu.sync_copy(ones, cnt_sh.at[segbuf], add=True)
        return carry

    lax.fori_loop(0, NB, macro_body, 0)
    plsc.subcore_barrier()
    half = NRP // 2
    for j in range(4):
        off = c * half + s * 12512 + j * 3128
        pltpu.sync_copy(cnt_sh.at[pl.ds(off, 3128)], zbuf.at[pl.ds(0, 3128)])
        pltpu.sync_copy(zbuf.at[pl.ds(0, 3128)], out_h.at[pl.ds(off, 3128)])


_hist = functools.partial(
    pl.kernel,
    out_type=jax.ShapeDtypeStruct((NRP,), jnp.float32),
    mesh=_mesh,
    compiler_params=_params,
    scratch_types=[
        pltpu.VMEM_SHARED((NRP,), jnp.float32),
        pltpu.VMEM((MACRO,), jnp.int32),
        pltpu.VMEM((MACRO,), jnp.int32),
        pltpu.VMEM((BATCH,), jnp.int32),
        pltpu.VMEM((BATCH,), jnp.float32),
        pltpu.VMEM((3200,), jnp.float32),
    ],
)(_hist_body)


WEP = E2 // 32
WNB = WEP // MACRO


def _wk_body(dst_h, et_h, cnt_h, out_h, dbuf, tbuf, segbuf, cbuf, wbuf, gsem):
    c = lax.axis_index("c")
    s = lax.axis_index("s")
    wid = c * NS + s

    def macro_body(k, carry):
        base = wid * WEP + k * MACRO
        pltpu.sync_copy(dst_h.at[pl.ds(base, MACRO)], dbuf)
        pltpu.sync_copy(et_h.at[pl.ds(base, MACRO)], tbuf)
        for b in range(NBB):
            for j in range(BATCH // 16):
                o = b * BATCH + j * 16
                dv = dbuf[pl.ds(o, 16)]
                tv = tbuf[pl.ds(o, 16)]
                segbuf[b, pl.ds(j * 16, 16)] = dv * R + tv
        descs = [pltpu.async_copy(cnt_h.at[segbuf.at[b]],
                                  cbuf.at[pl.ds(b * BATCH, BATCH)], gsem)
                 for b in range(NBB)]
        for d in descs:
            d.wait()
        for j in range(MACRO // 16):
            cv = cbuf[pl.ds(j * 16, 16)]
            wbuf[pl.ds(j * 16, 16)] = 1.0 / jnp.maximum(cv, 1.0)
        pltpu.sync_copy(wbuf, out_h.at[pl.ds(base, MACRO)])
        return carry

    lax.fori_loop(0, WNB, macro_body, 0)


_wk = functools.partial(
    pl.kernel,
    out_type=jax.ShapeDtypeStruct((E2,), jnp.float32),
    mesh=_mesh,
    compiler_params=_params,
    scratch_types=[
        pltpu.VMEM((MACRO,), jnp.int32),
        pltpu.VMEM((MACRO,), jnp.int32),
        pltpu.VMEM((NBB, BATCH), jnp.int32),
        pltpu.VMEM((MACRO,), jnp.float32),
        pltpu.VMEM((MACRO,), jnp.float32),
        pltpu.SemaphoreType.DMA,
    ],
)(_wk_body)


def _edge_body(src_h, et_h, dst_h, w_h, hflat_h, out_h,
               acc_sh, sbuf, tbuf, dbuf, wsbuf, gbuf, lbuf, wm, rows, zbuf,
               gsem, ssem, stsem):
    c = lax.axis_index("c")
    s = lax.axis_index("s")
    lo = c * NHALF

    def zfill(i, carry):
        for kk in range(4):
            zbuf[i, pl.ds(kk * 16, 16)] = jnp.zeros((16,), jnp.float32)
        return carry
    lax.fori_loop(0, 32, zfill, 0)

    def zcopy(j, carry):
        pltpu.sync_copy(zbuf, acc_sh.at[pl.ds(s * 1600 + j * 32, 32)])
        return carry
    lax.fori_loop(0, 50, zcopy, 0)
    plsc.subcore_barrier()

    def _stage(m, p, issue):
        base = s * EP + m * MACRO
        f = pltpu.async_copy if issue else pltpu.make_async_copy
        return [
            f(src_h.at[pl.ds(base, MACRO)], sbuf.at[p], stsem),
            f(et_h.at[pl.ds(base, MACRO)], tbuf.at[p], stsem),
            f(dst_h.at[pl.ds(base, MACRO)], dbuf.at[p], stsem),
            f(w_h.at[pl.ds(base, MACRO)], wsbuf.at[p], stsem),
        ]

    _stage(0, 0, True)

    def macro_body(m, carry):
        p = lax.rem(m, 2)
        for d in _stage(m, p, False):
            d.wait()

        @pl.when(m > 0)
        def _():
            for b in range(NBB):
                pltpu.make_async_copy(
                    rows.at[pl.ds(b * BATCH, BATCH)],
                    acc_sh.at[lbuf.at[b]], ssem).wait()

        for b in range(NBB):
            for j in range(BATCH // 16):
                o = b * BATCH + j * 16
                sv = sbuf[p, pl.ds(o, 16)]
                tv = tbuf[p, pl.ds(o, 16)]
                dv = dbuf[p, pl.ds(o, 16)]
                wv = wsbuf[p, pl.ds(o, 16)]
                gbuf[b, pl.ds(j * 16, 16)] = tv * N + sv
                m_in = (dv >= lo) & (dv < lo + NHALF)
                lbuf[b, pl.ds(j * 16, 16)] = jnp.where(m_in, dv - lo, DUMP)
                wm[pl.ds(o, 16)] = jnp.where(m_in, wv, 0.0)
        gd = [pltpu.async_copy(hflat_h.at[gbuf.at[b]],
                               rows.at[pl.ds(b * BATCH, BATCH)], gsem)
              for b in range(NBB)]

        @pl.when(m < NB - 1)
        def _():
            _stage(m + 1, 1 - p, True)

        for b in range(NBB):
            gd[b].wait()

            def mul_body(i, carry2):
                i4 = b * BATCH + i * 4
                for u in range(4):
                    wv = plsc.load_gather(
                        wm, [jnp.full((16,), i4 + u, jnp.int32)])
                    for kk in range(4):
                        rows[i4 + u, pl.ds(kk * 16, 16)] = (
                            rows[i4 + u, pl.ds(kk * 16, 16)] * wv)
                return carry2
            lax.fori_loop(0, BATCH // 4, mul_body, 0)
            pltpu.async_copy(rows.at[pl.ds(b * BATCH, BATCH)],
                             acc_sh.at[lbuf.at[b]], ssem, add=True)
        return carry

    lax.fori_loop(0, NB, macro_body, 0)
    for b in range(NBB):
        pltpu.make_async_copy(rows.at[pl.ds(b * BATCH, BATCH)],
                              acc_sh.at[lbuf.at[b]], ssem).wait()
    plsc.subcore_barrier()

    def wb(j, carry):
        pltpu.sync_copy(acc_sh.at[pl.ds(s * 1600 + j * 32, 32)], zbuf)
        pltpu.sync_copy(zbuf, out_h.at[pl.ds(c * ACC + s * 1600 + j * 32, 32)])
        return carry
    lax.fori_loop(0, 50, wb, 0)


_edge = functools.partial(
    pl.kernel,
    out_type=jax.ShapeDtypeStruct((2 * ACC, D), jnp.float32),
    mesh=_mesh,
    compiler_params=_params,
    scratch_types=[
        pltpu.VMEM_SHARED((ACC, D), jnp.float32),
        pltpu.VMEM((2, MACRO), jnp.int32),
        pltpu.VMEM((2, MACRO), jnp.int32),
        pltpu.VMEM((2, MACRO), jnp.int32),
        pltpu.VMEM((2, MACRO), jnp.float32),
        pltpu.VMEM((NBB, BATCH), jnp.int32),
        pltpu.VMEM((NBB, BATCH), jnp.int32),
        pltpu.VMEM((MACRO,), jnp.float32),
        pltpu.VMEM((MACRO, D), jnp.float32),
        pltpu.VMEM((32, D), jnp.float32),
        pltpu.SemaphoreType.DMA,
        pltpu.SemaphoreType.DMA,
        pltpu.SemaphoreType.DMA,
    ],
)(_edge_body)


def _mm_body(h_ref, w_ref, o_ref):
    h = h_ref[...]
    for r in range(R + 1):
        o_ref[r] = jnp.dot(h, w_ref[r], preferred_element_type=jnp.float32)


def _mm(h, wall):
    return pl.pallas_call(
        _mm_body,
        grid=(N // BLK,),
        in_specs=[
            pl.BlockSpec((BLK, D), lambda i: (i, 0)),
            pl.BlockSpec((R + 1, D, D), lambda i: (0, 0, 0)),
        ],
        out_specs=pl.BlockSpec((R + 1, BLK, D), lambda i: (0, i, 0)),
        out_shape=jax.ShapeDtypeStruct((R + 1, N, D), jnp.float32),
    )(h, wall)


def _combine_body_relu(a_ref, r_ref, b_ref, o_ref):
    o_ref[...] = jnp.maximum(a_ref[...] + r_ref[0] + b_ref[...], 0.0)


def _combine_body(a_ref, r_ref, b_ref, o_ref):
    o_ref[...] = a_ref[...] + r_ref[0] + b_ref[...]


def _combine(aggp, hall, b, relu):
    body = _combine_body_relu if relu else _combine_body
    return pl.pallas_call(
        body,
        grid=(N // CBLK,),
        in_specs=[
            pl.BlockSpec((CBLK, D), lambda i: (jnp.where(i < 125, i, i + 3), 0)),
            pl.BlockSpec((1, CBLK, D), lambda i: (R, i, 0)),
            pl.BlockSpec((1, D), lambda i: (0, 0)),
        ],
        out_specs=pl.BlockSpec((CBLK, D), lambda i: (i, 0)),
        out_shape=jax.ShapeDtypeStruct((N, D), jnp.float32),
    )(aggp, hall, b)


def kernel(x, edge_index, edge_type, emb, W1, root1, b1, W2, root2, b2):
    src = edge_index[0].astype(jnp.int32)
    dst = edge_index[1].astype(jnp.int32)
    et = edge_type.astype(jnp.int32)
    pad = E2 - E
    srcp = jnp.concatenate([src, jnp.zeros((pad,), jnp.int32)])
    etp = jnp.concatenate([et, jnp.zeros((pad,), jnp.int32)])
    dstp = jnp.concatenate([dst, jnp.full((pad,), N, jnp.int32)])
    h = jnp.take(emb, x, axis=0)
    cnt = _hist(dstp, etp)
    w = _wk(dstp, etp, cnt)
    for (W, root, b, relu) in ((W1, root1, b1, True), (W2, root2, b2, False)):
        wall = jnp.concatenate([W, root[None]], axis=0)
        hall = _mm(h, wall)
        aggp = _edge(srcp, etp, dstp, w, hall.reshape((R + 1) * N, D))
        h = _combine(aggp, hall, b.reshape(1, D), relu)
    return h
